# Initial kernel scaffold; baseline (speedup 1.0000x reference)
import jax
import jax.numpy as jnp
from jax.experimental import pallas as pl


def kernel(x, support0, support1, adj, W0, b0, Wl, Wc, bc):
    raise NotImplementedError("write your pallas kernel here")


# speedup vs baseline: 1.7463x; 1.7463x over previous
import numpy as np

import jax
import jax.numpy as jnp
from jax.experimental import pallas as pl

_NLAYERS = 4
_GAMMA = 0.5
_LAMDA = 0.5
_ALPHA = 0.1

_R1 = 80
_R2 = 400


def _mlp_body(x_ref, w_ref, b_ref, o_ref):
    y = jnp.dot(x_ref[...], w_ref[...], preferred_element_type=jnp.float32)
    o_ref[...] = jnp.maximum(y + b_ref[...], 0.0)


def _layer1_body(theta, s0_ref, s1_ref, hb_ref, h0_ref, w_ref, hn_ref, sc_ref):
    s = _GAMMA * s0_ref[...] + (1.0 - _GAMMA) * s1_ref[...]
    sb = s.astype(jnp.bfloat16)
    sc_ref[...] = sb
    acc = jnp.dot(sb, hb_ref[...], preferred_element_type=jnp.float32)
    support = (1.0 - _ALPHA) * acc + _ALPHA * h0_ref[...]
    y = jnp.dot(support, w_ref[...], preferred_element_type=jnp.float32)
    hn_ref[...] = jnp.maximum(theta * y + (1.0 - theta) * support, 0.0)


def _layer_body(theta, sc_ref, hb_ref, h0_ref, w_ref, hn_ref):
    acc = jnp.dot(sc_ref[...], hb_ref[...], preferred_element_type=jnp.float32)
    support = (1.0 - _ALPHA) * acc + _ALPHA * h0_ref[...]
    y = jnp.dot(support, w_ref[...], preferred_element_type=jnp.float32)
    hn_ref[...] = jnp.maximum(theta * y + (1.0 - theta) * support, 0.0)


def _head_layer_body(theta, sc_ref, hb_ref, h0_ref, w_ref, wc_ref, bc_ref,
                     o_ref):
    acc = jnp.dot(sc_ref[...], hb_ref[...], preferred_element_type=jnp.float32)
    support = (1.0 - _ALPHA) * acc + _ALPHA * h0_ref[...]
    y = jnp.dot(support, w_ref[...], preferred_element_type=jnp.float32)
    h = jnp.maximum(theta * y + (1.0 - theta) * support, 0.0)
    logits = jnp.dot(h, wc_ref[...], preferred_element_type=jnp.float32)
    logits = logits + bc_ref[...]
    m = jnp.max(logits, axis=1, keepdims=True)
    lse = m + jnp.log(jnp.sum(jnp.exp(logits - m), axis=1, keepdims=True))
    o_ref[...] = logits - lse


def _theta(layer_idx):
    return float(np.log(_LAMDA / (layer_idx + 1) + 1.0))


def kernel(x, support0, support1, adj, W0, b0, Wl, Wc, bc):
    del adj
    n, nfeat = x.shape
    nhid = W0.shape[1]
    nclass = Wc.shape[1]

    h = pl.pallas_call(
        _mlp_body,
        out_shape=jax.ShapeDtypeStruct((n, nhid), jnp.float32),
    )(x, W0, b0.reshape(1, nhid))

    h0 = h
    hb = h.astype(jnp.bfloat16)

    g1 = n // _R1
    h, sc = pl.pallas_call(
        lambda *refs: _layer1_body(_theta(0), *refs),
        grid=(g1,),
        in_specs=[
            pl.BlockSpec((_R1, n), lambda i: (i, 0)),
            pl.BlockSpec((_R1, n), lambda i: (i, 0)),
            pl.BlockSpec((n, nhid), lambda i: (0, 0)),
            pl.BlockSpec((_R1, nhid), lambda i: (i, 0)),
            pl.BlockSpec((nhid, nhid), lambda i: (0, 0)),
        ],
        out_specs=[
            pl.BlockSpec((_R1, nhid), lambda i: (i, 0)),
            pl.BlockSpec((_R1, n), lambda i: (i, 0)),
        ],
        out_shape=[
            jax.ShapeDtypeStruct((n, nhid), jnp.float32),
            jax.ShapeDtypeStruct((n, n), jnp.bfloat16),
        ],
    )(support0, support1, hb, h0, Wl[0])

    g2 = n // _R2
    common_in_specs = [
        pl.BlockSpec((_R2, n), lambda i: (i, 0)),
        pl.BlockSpec((n, nhid), lambda i: (0, 0)),
        pl.BlockSpec((_R2, nhid), lambda i: (i, 0)),
        pl.BlockSpec((nhid, nhid), lambda i: (0, 0)),
    ]
    for i in range(1, _NLAYERS - 1):
        hb = h.astype(jnp.bfloat16)
        th = _theta(i)
        h = pl.pallas_call(
            lambda *refs, _th=th: _layer_body(_th, *refs),
            grid=(g2,),
            in_specs=common_in_specs,
            out_specs=pl.BlockSpec((_R2, nhid), lambda i: (i, 0)),
            out_shape=jax.ShapeDtypeStruct((n, nhid), jnp.float32),
        )(sc, hb, h0, Wl[i])

    hb = h.astype(jnp.bfloat16)
    out = pl.pallas_call(
        lambda *refs: _head_layer_body(_theta(_NLAYERS - 1), *refs),
        grid=(g2,),
        in_specs=common_in_specs + [
            pl.BlockSpec((nhid, nclass), lambda i: (0, 0)),
            pl.BlockSpec((1, nclass), lambda i: (0, 0)),
        ],
        out_specs=pl.BlockSpec((_R2, nclass), lambda i: (i, 0)),
        out_shape=jax.ShapeDtypeStruct((n, nclass), jnp.float32),
    )(sc, hb, h0, Wl[_NLAYERS - 1], Wc, bc.reshape(1, nclass))
    return out


# speedup vs baseline: 2.0114x; 1.1518x over previous
import numpy as np

import jax
import jax.numpy as jnp
from jax.experimental import pallas as pl

_NLAYERS = 4
_GAMMA = 0.5
_LAMDA = 0.5
_ALPHA = 0.1

_R1 = 96
_R2 = 512
_SSCALE = 4096.0


def _mlp_body(x_ref, w_ref, b_ref, o_ref):
    y = jnp.dot(x_ref[...], w_ref[...], preferred_element_type=jnp.float32)
    o_ref[...] = jnp.maximum(y + b_ref[...], 0.0)


def _layer1_body(theta, s0_ref, s1_ref, hb_ref, h0_ref, w_ref, hn_ref, sc_ref):
    s = _GAMMA * s0_ref[...] + (1.0 - _GAMMA) * s1_ref[...]
    sc_ref[...] = (s * _SSCALE).astype(jnp.float8_e4m3fn)
    acc = jnp.dot(s.astype(jnp.bfloat16), hb_ref[...],
                  preferred_element_type=jnp.float32)
    support = (1.0 - _ALPHA) * acc + _ALPHA * h0_ref[...]
    y = jnp.dot(support, w_ref[...], preferred_element_type=jnp.float32)
    hn_ref[...] = jnp.maximum(theta * y + (1.0 - theta) * support, 0.0)


def _layer_body(theta, sc_ref, hb_ref, h0_ref, w_ref, hn_ref):
    scb = sc_ref[...].astype(jnp.bfloat16)
    acc = jnp.dot(scb, hb_ref[...], preferred_element_type=jnp.float32)
    support = ((1.0 - _ALPHA) / _SSCALE) * acc + _ALPHA * h0_ref[...]
    y = jnp.dot(support, w_ref[...], preferred_element_type=jnp.float32)
    hn_ref[...] = jnp.maximum(theta * y + (1.0 - theta) * support, 0.0)


def _head_layer_body(theta, sc_ref, hb_ref, h0_ref, w_ref, wc_ref, bc_ref,
                     o_ref):
    scb = sc_ref[...].astype(jnp.bfloat16)
    acc = jnp.dot(scb, hb_ref[...], preferred_element_type=jnp.float32)
    support = ((1.0 - _ALPHA) / _SSCALE) * acc + _ALPHA * h0_ref[...]
    y = jnp.dot(support, w_ref[...], preferred_element_type=jnp.float32)
    h = jnp.maximum(theta * y + (1.0 - theta) * support, 0.0)
    logits = jnp.dot(h, wc_ref[...], preferred_element_type=jnp.float32)
    logits = logits + bc_ref[...]
    m = jnp.max(logits, axis=1, keepdims=True)
    lse = m + jnp.log(jnp.sum(jnp.exp(logits - m), axis=1, keepdims=True))
    o_ref[...] = logits - lse


def _theta(layer_idx):
    return float(np.log(_LAMDA / (layer_idx + 1) + 1.0))


def kernel(x, support0, support1, adj, W0, b0, Wl, Wc, bc):
    del adj
    n, nfeat = x.shape
    nhid = W0.shape[1]
    nclass = Wc.shape[1]

    h = pl.pallas_call(
        _mlp_body,
        out_shape=jax.ShapeDtypeStruct((n, nhid), jnp.float32),
    )(x, W0, b0.reshape(1, nhid))

    h0 = h
    hb = h.astype(jnp.bfloat16)

    g1 = pl.cdiv(n, _R1)
    h, sc = pl.pallas_call(
        lambda *refs: _layer1_body(_theta(0), *refs),
        grid=(g1,),
        in_specs=[
            pl.BlockSpec((_R1, n), lambda i: (i, 0)),
            pl.BlockSpec((_R1, n), lambda i: (i, 0)),
            pl.BlockSpec((n, nhid), lambda i: (0, 0)),
            pl.BlockSpec((_R1, nhid), lambda i: (i, 0)),
            pl.BlockSpec((nhid, nhid), lambda i: (0, 0)),
        ],
        out_specs=[
            pl.BlockSpec((_R1, nhid), lambda i: (i, 0)),
            pl.BlockSpec((_R1, n), lambda i: (i, 0)),
        ],
        out_shape=[
            jax.ShapeDtypeStruct((n, nhid), jnp.float32),
            jax.ShapeDtypeStruct((n, n), jnp.float8_e4m3fn),
        ],
    )(support0, support1, hb, h0, Wl[0])

    g2 = pl.cdiv(n, _R2)
    common_in_specs = [
        pl.BlockSpec((_R2, n), lambda i: (i, 0)),
        pl.BlockSpec((n, nhid), lambda i: (0, 0)),
        pl.BlockSpec((_R2, nhid), lambda i: (i, 0)),
        pl.BlockSpec((nhid, nhid), lambda i: (0, 0)),
    ]
    for i in range(1, _NLAYERS - 1):
        hb = h.astype(jnp.bfloat16)
        th = _theta(i)
        h = pl.pallas_call(
            lambda *refs, _th=th: _layer_body(_th, *refs),
            grid=(g2,),
            in_specs=common_in_specs,
            out_specs=pl.BlockSpec((_R2, nhid), lambda i: (i, 0)),
            out_shape=jax.ShapeDtypeStruct((n, nhid), jnp.float32),
        )(sc, hb, h0, Wl[i])

    hb = h.astype(jnp.bfloat16)
    out = pl.pallas_call(
        lambda *refs: _head_layer_body(_theta(_NLAYERS - 1), *refs),
        grid=(g2,),
        in_specs=common_in_specs + [
            pl.BlockSpec((nhid, nclass), lambda i: (0, 0)),
            pl.BlockSpec((1, nclass), lambda i: (0, 0)),
        ],
        out_specs=pl.BlockSpec((_R2, nclass), lambda i: (i, 0)),
        out_shape=jax.ShapeDtypeStruct((n, nclass), jnp.float32),
    )(sc, hb, h0, Wl[_NLAYERS - 1], Wc, bc.reshape(1, nclass))
    return out


# speedup vs baseline: 2.3515x; 1.1691x over previous
import numpy as np

import jax
import jax.numpy as jnp
from jax.experimental import pallas as pl

_NLAYERS = 4
_GAMMA = 0.5
_LAMDA = 0.5
_ALPHA = 0.1

_R1 = 96
_R2 = 512
_SSCALE = 4096.0


def _mlp_body(x_ref, w_ref, b_ref, o_ref):
    y = jnp.dot(x_ref[...], w_ref[...], preferred_element_type=jnp.float32)
    o_ref[...] = jnp.maximum(y + b_ref[...], 0.0)


def _layer1_body(theta, s0_ref, s1_ref, hb_ref, h0_ref, w_ref, hn_ref, sc_ref):
    s = _GAMMA * s0_ref[...] + (1.0 - _GAMMA) * s1_ref[...]
    sc_ref[...] = (s * _SSCALE).astype(jnp.float8_e4m3fn)
    acc = jnp.dot(s.astype(jnp.bfloat16), hb_ref[...],
                  preferred_element_type=jnp.float32)
    support = (1.0 - _ALPHA) * acc + _ALPHA * h0_ref[...]
    y = jnp.dot(support, w_ref[...], preferred_element_type=jnp.float32)
    hn_ref[...] = jnp.maximum(theta * y + (1.0 - theta) * support, 0.0)


def _layer_body(theta, sc_ref, hb_ref, h0_ref, w_ref, hn_ref):
    acc = jnp.dot(sc_ref[...], hb_ref[...], preferred_element_type=jnp.float32)
    support = ((1.0 - _ALPHA) / _SSCALE) * acc + _ALPHA * h0_ref[...]
    y = jnp.dot(support, w_ref[...], preferred_element_type=jnp.float32)
    hn_ref[...] = jnp.maximum(theta * y + (1.0 - theta) * support, 0.0)


def _head_layer_body(theta, sc_ref, hb_ref, h0_ref, w_ref, wc_ref, bc_ref,
                     o_ref):
    acc = jnp.dot(sc_ref[...], hb_ref[...], preferred_element_type=jnp.float32)
    support = ((1.0 - _ALPHA) / _SSCALE) * acc + _ALPHA * h0_ref[...]
    y = jnp.dot(support, w_ref[...], preferred_element_type=jnp.float32)
    h = jnp.maximum(theta * y + (1.0 - theta) * support, 0.0)
    logits = jnp.dot(h, wc_ref[...], preferred_element_type=jnp.float32)
    logits = logits + bc_ref[...]
    m = jnp.max(logits, axis=1, keepdims=True)
    lse = m + jnp.log(jnp.sum(jnp.exp(logits - m), axis=1, keepdims=True))
    o_ref[...] = logits - lse


def _theta(layer_idx):
    return float(np.log(_LAMDA / (layer_idx + 1) + 1.0))


def kernel(x, support0, support1, adj, W0, b0, Wl, Wc, bc):
    del adj
    n, nfeat = x.shape
    nhid = W0.shape[1]
    nclass = Wc.shape[1]

    h = pl.pallas_call(
        _mlp_body,
        out_shape=jax.ShapeDtypeStruct((n, nhid), jnp.float32),
    )(x, W0, b0.reshape(1, nhid))

    h0 = h
    hb = h.astype(jnp.bfloat16)

    g1 = pl.cdiv(n, _R1)
    h, sc = pl.pallas_call(
        lambda *refs: _layer1_body(_theta(0), *refs),
        grid=(g1,),
        in_specs=[
            pl.BlockSpec((_R1, n), lambda i: (i, 0)),
            pl.BlockSpec((_R1, n), lambda i: (i, 0)),
            pl.BlockSpec((n, nhid), lambda i: (0, 0)),
            pl.BlockSpec((_R1, nhid), lambda i: (i, 0)),
            pl.BlockSpec((nhid, nhid), lambda i: (0, 0)),
        ],
        out_specs=[
            pl.BlockSpec((_R1, nhid), lambda i: (i, 0)),
            pl.BlockSpec((_R1, n), lambda i: (i, 0)),
        ],
        out_shape=[
            jax.ShapeDtypeStruct((n, nhid), jnp.float32),
            jax.ShapeDtypeStruct((n, n), jnp.float8_e4m3fn),
        ],
    )(support0, support1, hb, h0, Wl[0])

    g2 = pl.cdiv(n, _R2)
    common_in_specs = [
        pl.BlockSpec((_R2, n), lambda i: (i, 0)),
        pl.BlockSpec((n, nhid), lambda i: (0, 0)),
        pl.BlockSpec((_R2, nhid), lambda i: (i, 0)),
        pl.BlockSpec((nhid, nhid), lambda i: (0, 0)),
    ]
    for i in range(1, _NLAYERS - 1):
        hb = h.astype(jnp.float8_e4m3fn)
        th = _theta(i)
        h = pl.pallas_call(
            lambda *refs, _th=th: _layer_body(_th, *refs),
            grid=(g2,),
            in_specs=common_in_specs,
            out_specs=pl.BlockSpec((_R2, nhid), lambda i: (i, 0)),
            out_shape=jax.ShapeDtypeStruct((n, nhid), jnp.float32),
        )(sc, hb, h0, Wl[i])

    hb = h.astype(jnp.float8_e4m3fn)
    out = pl.pallas_call(
        lambda *refs: _head_layer_body(_theta(_NLAYERS - 1), *refs),
        grid=(g2,),
        in_specs=common_in_specs + [
            pl.BlockSpec((nhid, nclass), lambda i: (0, 0)),
            pl.BlockSpec((1, nclass), lambda i: (0, 0)),
        ],
        out_specs=pl.BlockSpec((_R2, nclass), lambda i: (i, 0)),
        out_shape=jax.ShapeDtypeStruct((n, nclass), jnp.float32),
    )(sc, hb, h0, Wl[_NLAYERS - 1], Wc, bc.reshape(1, nclass))
    return out
